# Initial kernel scaffold; baseline (speedup 1.0000x reference)
import jax
import jax.numpy as jnp
from jax.experimental import pallas as pl


def kernel(pos, x, offset, wq, bq, wk, bk, wv, bv, pos_w1, pos_b1, pos_bn_g, pos_bn_b, pos_w2, pos_b2, mlp_bn1_g, mlp_bn1_b, mlp_w1, mlp_b1, mlp_bn2_g, mlp_bn2_b, mlp_w2, mlp_b2):
    raise NotImplementedError("write your pallas kernel here")


# speedup vs baseline: 7.0703x; 7.0703x over previous
import functools

import jax
import jax.numpy as jnp
from jax import lax
from jax.experimental import pallas as pl
from jax.experimental.pallas import tpu as pltpu
from jax.experimental.pallas import tpu_sc as plsc

N = 16384
B = 8
NB = N // B
C = 64
K = 16
H = 8
DH = C // H
ROWS = N * K
CNT = float(ROWS)
EPS = 1e-5

F32 = jnp.float32
HI = lax.Precision.HIGHEST


_RB1 = 1024


def _qkv_body(x_ref, wqT, bq, wkT, bk, wvT, bv, q_ref, k_ref, v_ref):
    xb = x_ref[...]
    q_ref[...] = jnp.dot(xb, wqT[...], preferred_element_type=F32) + bq[...]
    k_ref[...] = jnp.dot(xb, wkT[...], preferred_element_type=F32) + bk[...]
    v_ref[...] = jnp.dot(xb, wvT[...], preferred_element_type=F32) + bv[...]


def _qkv(x, wqT, bq, wkT, bk, wvT, bv):
    full = lambda shp: pl.BlockSpec(shp, lambda i: (0,) * len(shp))
    return pl.pallas_call(
        _qkv_body,
        grid=(N // _RB1,),
        in_specs=[
            pl.BlockSpec((_RB1, C), lambda i: (i, 0)),
            full((C, C)), full((1, C)), full((C, C)), full((1, C)),
            full((C, C)), full((1, C)),
        ],
        out_specs=[pl.BlockSpec((_RB1, C), lambda i: (i, 0))] * 3,
        out_shape=[jax.ShapeDtypeStruct((N, C), F32)] * 3,
    )(x, wqT, bq, wkT, bk, wvT, bv)


_QB = 256


def _knn_body(posq_ref, posTb_ref, idx_ref):
    pq = posq_ref[...]
    ct = posTb_ref[...]
    sqq = jnp.sum(pq * pq, axis=1, keepdims=True)
    sqc = jnp.sum(ct * ct, axis=0, keepdims=True)
    dot = jnp.dot(pq, ct, preferred_element_type=F32)
    d = (sqq + sqc) - 2.0 * dot
    iota = lax.broadcasted_iota(jnp.int32, d.shape, 1)
    iota16 = lax.broadcasted_iota(jnp.int32, (_QB, K), 1)
    big = F32(3e38)
    bigi = jnp.int32(NB)
    idx_acc = jnp.zeros((_QB, K), jnp.int32)
    for t in range(K):
        m = jnp.min(d, axis=1, keepdims=True)
        am = jnp.min(jnp.where(d <= m, iota, bigi), axis=1, keepdims=True)
        d = jnp.where(iota == am, big, d)
        idx_acc = idx_acc + jnp.where(iota16 == t, am, 0)
    b = pl.program_id(0)
    idx_ref[...] = idx_acc + b * NB


def _knn(pos, posT):
    return pl.pallas_call(
        _knn_body,
        grid=(B, NB // _QB),
        in_specs=[
            pl.BlockSpec((_QB, 3), lambda b, q: (b * (NB // _QB) + q, 0)),
            pl.BlockSpec((3, NB), lambda b, q: (0, b)),
        ],
        out_specs=pl.BlockSpec((_QB, K), lambda b, q: (b * (NB // _QB) + q, 0)),
        out_shape=jax.ShapeDtypeStruct((N, K), jnp.int32),
    )(pos, posT)


_NW = 32
_PER_W = ROWS // _NW
_CH = 128
_NCH = _PER_W // _CH


@functools.lru_cache(maxsize=1)
def _build_sc_gather():
    mesh = plsc.VectorSubcoreMesh(core_axis_name="c", subcore_axis_name="s")

    @functools.partial(
        pl.kernel,
        mesh=mesh,
        compiler_params=pltpu.CompilerParams(use_tc_tiling_on_sc=False),
        out_type=(
            jax.ShapeDtypeStruct((ROWS, C), F32),
            jax.ShapeDtypeStruct((ROWS, C), F32),
            jax.ShapeDtypeStruct((ROWS, 16), F32),
        ),
        scratch_types=[
            pltpu.VMEM((_CH,), jnp.int32),
            pltpu.VMEM((_CH, C), F32),
            pltpu.VMEM((_CH, C), F32),
            pltpu.VMEM((_CH, 16), F32),
            pltpu.SemaphoreType.DMA,
            pltpu.SemaphoreType.DMA,
            pltpu.SemaphoreType.DMA,
        ],
    )
    def gather_k(idx_hbm, wk_hbm, wv_hbm, pp_hbm, kg_hbm, vg_hbm, pg_hbm,
                 idxb, kb, vb, pb, semk, semv, semp):
        wid = lax.axis_index("s") * 2 + lax.axis_index("c")

        def body(j, carry):
            base = wid * _PER_W + j * _CH
            pltpu.sync_copy(idx_hbm.at[pl.ds(base, _CH)], idxb)
            ck = pltpu.async_copy(wk_hbm.at[idxb], kb, semk)
            cv = pltpu.async_copy(wv_hbm.at[idxb], vb, semv)
            cp = pltpu.async_copy(pp_hbm.at[idxb], pb, semp)
            ck.wait()
            cv.wait()
            cp.wait()
            pltpu.sync_copy(kb, kg_hbm.at[pl.ds(base, _CH)])
            pltpu.sync_copy(vb, vg_hbm.at[pl.ds(base, _CH)])
            pltpu.sync_copy(pb, pg_hbm.at[pl.ds(base, _CH)])
            return carry

        lax.fori_loop(0, _NCH, body, 0)

    return gather_k


def _sc_gather(idxf, wk, wv, pp):
    return _build_sc_gather()(idxf, wk, wv, pp)


_PB = 128
_RB = _PB * K


def _seg_expand(rb, nb):
    ri = lax.broadcasted_iota(jnp.int32, (rb, nb), 0)
    ni = lax.broadcasted_iota(jnp.int32, (rb, nb), 1)
    return jnp.where(ri // K == ni, F32(1.0), F32(0.0))


def _seg_reduce(nb, rb):
    ni = lax.broadcasted_iota(jnp.int32, (nb, rb), 0)
    ri = lax.broadcasted_iota(jnp.int32, (nb, rb), 1)
    return jnp.where(ri // K == ni, F32(1.0), F32(0.0))


def _mom_pad(s0, s1, width):
    z = jnp.zeros((1, 128 - width), F32)
    r0 = jnp.concatenate([s0, z], axis=1)
    r1 = jnp.concatenate([s1, z], axis=1)
    blk = jnp.concatenate([r0, r1, jnp.zeros((6, 128), F32)], axis=0)
    return blk.reshape(1, 8, 128)


def _stats(mom_arr, width):
    moms = jnp.sum(mom_arr, axis=0)
    mean = moms[0:1, 0:width] / CNT
    var = moms[1:2, 0:width] / CNT - mean * mean
    return mean, lax.rsqrt(var + EPS)


def _pos_h(pg, pp_blk, st, w1p, b1p):
    pp_rep = jnp.dot(st, pp_blk, precision=HI, preferred_element_type=F32)
    rel = pg - pp_rep
    return jnp.dot(rel, w1p, preferred_element_type=F32) + b1p


def _pos_enc(h, mean1, rstd1, g1p, bt1p, w2p, b2p):
    hn = (h - mean1) * rstd1 * g1p + bt1p
    r = jnp.maximum(hn, 0.0)
    return jnp.dot(r, w2p, preferred_element_type=F32) + b2p


def _relmom_body(pg_ref, pp_ref, w1p, b1p, mom_ref):
    st = _seg_expand(_RB, _PB)
    h = _pos_h(pg_ref[...], pp_ref[...], st, w1p[...], b1p[...])
    s0 = jnp.sum(h, axis=0, keepdims=True)
    s1 = jnp.sum(h * h, axis=0, keepdims=True)
    mom_ref[...] = _mom_pad(s0, s1, 16)


def _relmom(pg, pp, w1p, b1p):
    full = lambda shp: pl.BlockSpec(shp, lambda i: (0,) * len(shp))
    nblk = N // _PB
    return pl.pallas_call(
        _relmom_body,
        grid=(nblk,),
        in_specs=[
            pl.BlockSpec((_RB, 16), lambda i: (i, 0)),
            pl.BlockSpec((_PB, 16), lambda i: (i, 0)),
            full((16, 16)), full((1, 16)),
        ],
        out_specs=pl.BlockSpec((1, 8, 128), lambda i: (i, 0, 0)),
        out_shape=jax.ShapeDtypeStruct((nblk, 8, 128), F32),
    )(pg, pp, w1p, b1p)


def _passB_body(kg_ref, pg_ref, wq_ref, pp_ref, mom1_ref, w1p, b1p, g1p, bt1p,
                w2p, b2p, attn_ref, mom_ref):
    st = _seg_expand(_RB, _PB)
    mean1, rstd1 = _stats(mom1_ref[...], 16)
    h = _pos_h(pg_ref[...], pp_ref[...], st, w1p[...], b1p[...])
    pe = _pos_enc(h, mean1, rstd1, g1p[...], bt1p[...], w2p[...], b2p[...])
    wq_rep = jnp.dot(st, wq_ref[...], precision=HI, preferred_element_type=F32)
    a = kg_ref[...] - wq_rep + pe
    attn_ref[...] = a
    s0 = jnp.sum(a, axis=0, keepdims=True)
    s1 = jnp.sum(a * a, axis=0, keepdims=True)
    mom_ref[...] = _mom_pad(s0, s1, C)


def _passB(kg, pg, wq_full, pp, mom1, w1p, b1p, g1p, bt1p, w2p, b2p):
    full = lambda shp: pl.BlockSpec(shp, lambda i: (0,) * len(shp))
    nblk = N // _PB
    return pl.pallas_call(
        _passB_body,
        grid=(nblk,),
        in_specs=[
            pl.BlockSpec((_RB, C), lambda i: (i, 0)),
            pl.BlockSpec((_RB, 16), lambda i: (i, 0)),
            pl.BlockSpec((_PB, C), lambda i: (i, 0)),
            pl.BlockSpec((_PB, 16), lambda i: (i, 0)),
            full((nblk, 8, 128)),
            full((16, 16)), full((1, 16)), full((1, 16)), full((1, 16)),
            full((16, C)), full((1, C)),
        ],
        out_specs=[
            pl.BlockSpec((_RB, C), lambda i: (i, 0)),
            pl.BlockSpec((1, 8, 128), lambda i: (i, 0, 0)),
        ],
        out_shape=[
            jax.ShapeDtypeStruct((ROWS, C), F32),
            jax.ShapeDtypeStruct((nblk, 8, 128), F32),
        ],
    )(kg, pg, wq_full, pp, mom1, w1p, b1p, g1p, bt1p, w2p, b2p)


_RBC = 8192


def _passC_body(attn_ref, mom2_ref, w1T, b1m, g1m, bt1m, s_ref, mom_ref):
    mean2, rstd2 = _stats(mom2_ref[...], C)
    an = (attn_ref[...] - mean2) * rstd2 * g1m[...] + bt1m[...]
    r = jnp.maximum(an, 0.0)
    s = jnp.dot(r, w1T[...], preferred_element_type=F32) + b1m[...]
    s_ref[...] = s
    s0 = jnp.sum(s, axis=0, keepdims=True)
    s1 = jnp.sum(s * s, axis=0, keepdims=True)
    mom_ref[...] = _mom_pad(s0, s1, DH)


def _passC(attn, mom2, w1T, b1m, g1m, bt1m):
    full = lambda shp: pl.BlockSpec(shp, lambda i: (0,) * len(shp))
    nblk = ROWS // _RBC
    nblkB = N // _PB
    return pl.pallas_call(
        _passC_body,
        grid=(nblk,),
        in_specs=[
            pl.BlockSpec((_RBC, C), lambda i: (i, 0)),
            full((nblkB, 8, 128)),
            full((C, DH)), full((1, DH)), full((1, C)), full((1, C)),
        ],
        out_specs=[
            pl.BlockSpec((_RBC, DH), lambda i: (i, 0)),
            pl.BlockSpec((1, 8, 128), lambda i: (i, 0, 0)),
        ],
        out_shape=[
            jax.ShapeDtypeStruct((ROWS, DH), F32),
            jax.ShapeDtypeStruct((nblk, 8, 128), F32),
        ],
    )(attn, mom2, w1T, b1m, g1m, bt1m)


def _passD_body(s_ref, mom3_ref, vg_ref, pg_ref, pp_ref, mom1_ref,
                w1p, b1p, g1p, bt1p, w2p, b2p,
                w2T, b2m, g2m, bt2m, t8, out_ref):
    mean3, rstd3 = _stats(mom3_ref[...], DH)
    sn = (s_ref[...] - mean3) * rstd3 * g2m[...] + bt2m[...]
    u = jnp.dot(jnp.maximum(sn, 0.0), w2T[...],
                preferred_element_type=F32) + b2m[...]
    e = jnp.exp(u)
    ssel = _seg_reduce(_PB, _RB)
    st = _seg_expand(_RB, _PB)
    den = jnp.dot(ssel, e, precision=HI, preferred_element_type=F32)
    den_rep = jnp.dot(st, den, precision=HI, preferred_element_type=F32)
    w = e / den_rep
    mean1, rstd1 = _stats(mom1_ref[...], 16)
    h = _pos_h(pg_ref[...], pp_ref[...], st, w1p[...], b1p[...])
    pe = _pos_enc(h, mean1, rstd1, g1p[...], bt1p[...], w2p[...], b2p[...])
    v = vg_ref[...] + pe
    w64 = jnp.dot(w, t8[...], precision=HI, preferred_element_type=F32)
    out_ref[...] = jnp.dot(ssel, v * w64, precision=HI,
                           preferred_element_type=F32)


def _passD(s, mom3, vg, pg, pp, mom1, w1p, b1p, g1p, bt1p, w2p, b2p,
           w2T, b2m, g2m, bt2m, t8):
    full = lambda shp: pl.BlockSpec(shp, lambda i: (0,) * len(shp))
    nblk = N // _PB
    nblkC = ROWS // _RBC
    return pl.pallas_call(
        _passD_body,
        grid=(nblk,),
        in_specs=[
            pl.BlockSpec((_RB, DH), lambda i: (i, 0)),
            full((nblkC, 8, 128)),
            pl.BlockSpec((_RB, C), lambda i: (i, 0)),
            pl.BlockSpec((_RB, 16), lambda i: (i, 0)),
            pl.BlockSpec((_PB, 16), lambda i: (i, 0)),
            full((nblk, 8, 128)),
            full((16, 16)), full((1, 16)), full((1, 16)), full((1, 16)),
            full((16, C)), full((1, C)),
            full((DH, DH)), full((1, DH)), full((1, DH)), full((1, DH)),
            full((DH, C)),
        ],
        out_specs=pl.BlockSpec((_PB, C), lambda i: (i, 0)),
        out_shape=jax.ShapeDtypeStruct((N, C), F32),
    )(s, mom3, vg, pg, pp, mom1, w1p, b1p, g1p, bt1p, w2p, b2p,
      w2T, b2m, g2m, bt2m, t8)


def kernel(pos, x, offset, wq, bq, wk, bk, wv, bv, pos_w1, pos_b1, pos_bn_g,
           pos_bn_b, pos_w2, pos_b2, mlp_bn1_g, mlp_bn1_b, mlp_w1, mlp_b1,
           mlp_bn2_g, mlp_bn2_b, mlp_w2, mlp_b2):
    posT = pos.T
    pp = jnp.pad(pos, ((0, 0), (0, 13)))
    pad16 = lambda v: jnp.pad(v, (0, 13)).reshape(1, 16)
    w1p = jnp.pad(pos_w1.T, ((0, 13), (0, 13)))
    b1p = pad16(pos_b1)
    g1p = pad16(pos_bn_g)
    bt1p = pad16(pos_bn_b)
    w2p = jnp.pad(pos_w2.T, ((0, 13), (0, 0)))
    b2p = pos_b2.reshape(1, C)
    w1T = mlp_w1.T
    b1m = mlp_b1.reshape(1, DH)
    g1m = mlp_bn1_g.reshape(1, C)
    bt1m = mlp_bn1_b.reshape(1, C)
    w2T = mlp_w2.T
    b2m = mlp_b2.reshape(1, DH)
    g2m = mlp_bn2_g.reshape(1, DH)
    bt2m = mlp_bn2_b.reshape(1, DH)
    t8 = jnp.tile(jnp.eye(DH, dtype=F32), (1, H))

    wq_full, wk_full, wv_full = _qkv(x, wq.T, bq.reshape(1, C), wk.T,
                                     bk.reshape(1, C), wv.T, bv.reshape(1, C))
    idx = _knn(pos, posT)
    kg, vg, pg = _sc_gather(idx.reshape(ROWS), wk_full, wv_full, pp)
    mom1 = _relmom(pg, pp, w1p, b1p)
    attn, mom2 = _passB(kg, pg, wq_full, pp, mom1, w1p, b1p, g1p, bt1p,
                        w2p, b2p)
    s, mom3 = _passC(attn, mom2, w1T, b1m, g1m, bt1m)
    return _passD(s, mom3, vg, pg, pp, mom1, w1p, b1p, g1p, bt1p, w2p, b2p,
                  w2T, b2m, g2m, bt2m, t8)


# speedup vs baseline: 11.6834x; 1.6525x over previous
import functools

import jax
import jax.numpy as jnp
from jax import lax
from jax.experimental import pallas as pl
from jax.experimental.pallas import tpu as pltpu
from jax.experimental.pallas import tpu_sc as plsc

N = 16384
B = 8
NB = N // B
C = 64
K = 16
H = 8
DH = C // H
ROWS = N * K
CNT = float(ROWS)
EPS = 1e-5

F32 = jnp.float32
HI = lax.Precision.HIGHEST


_RB1 = 1024


def _qkv_body(x_ref, wqT, bq, wkT, bk, wvT, bv, q_ref, k_ref, v_ref):
    xb = x_ref[...]
    q_ref[...] = jnp.dot(xb, wqT[...], preferred_element_type=F32) + bq[...]
    k_ref[...] = jnp.dot(xb, wkT[...], preferred_element_type=F32) + bk[...]
    v_ref[...] = jnp.dot(xb, wvT[...], preferred_element_type=F32) + bv[...]


def _qkv(x, wqT, bq, wkT, bk, wvT, bv):
    full = lambda shp: pl.BlockSpec(shp, lambda i: (0,) * len(shp))
    return pl.pallas_call(
        _qkv_body,
        grid=(N // _RB1,),
        in_specs=[
            pl.BlockSpec((_RB1, C), lambda i: (i, 0)),
            full((C, C)), full((1, C)), full((C, C)), full((1, C)),
            full((C, C)), full((1, C)),
        ],
        out_specs=[pl.BlockSpec((_RB1, C), lambda i: (i, 0))] * 3,
        out_shape=[jax.ShapeDtypeStruct((N, C), F32)] * 3,
    )(x, wqT, bq, wkT, bk, wvT, bv)


_QB = 256


def _knn_body(posq_ref, posTb_ref, idx_ref):
    pq = posq_ref[...]
    ct = posTb_ref[...]
    sqq = jnp.sum(pq * pq, axis=1, keepdims=True)
    sqc = jnp.sum(ct * ct, axis=0, keepdims=True)
    dot = jnp.dot(pq, ct, preferred_element_type=F32)
    d = (sqq + sqc) - 2.0 * dot
    iota = lax.broadcasted_iota(jnp.int32, d.shape, 1)
    iota16 = lax.broadcasted_iota(jnp.int32, (_QB, K), 1)
    big = F32(3e38)
    bigi = jnp.int32(NB)
    del bigi
    idx_acc = jnp.zeros((_QB, K), jnp.int32)
    for t in range(K):
        am = jnp.argmin(d, axis=1).reshape(_QB, 1).astype(jnp.int32)
        d = jnp.where(iota == am, big, d)
        idx_acc = idx_acc + jnp.where(iota16 == t, am, 0)
    b = pl.program_id(0)
    idx_ref[...] = idx_acc + b * NB


def _knn(pos, posT):
    return pl.pallas_call(
        _knn_body,
        grid=(B, NB // _QB),
        in_specs=[
            pl.BlockSpec((_QB, 3), lambda b, q: (b * (NB // _QB) + q, 0)),
            pl.BlockSpec((3, NB), lambda b, q: (0, b)),
        ],
        out_specs=pl.BlockSpec((_QB, K), lambda b, q: (b * (NB // _QB) + q, 0)),
        out_shape=jax.ShapeDtypeStruct((N, K), jnp.int32),
    )(pos, posT)


_NW = 32
_PER_W = ROWS // _NW
_CH = 128
_NCH = _PER_W // _CH


@functools.lru_cache(maxsize=1)
def _build_sc_gather():
    mesh = plsc.VectorSubcoreMesh(core_axis_name="c", subcore_axis_name="s")

    @functools.partial(
        pl.kernel,
        mesh=mesh,
        compiler_params=pltpu.CompilerParams(use_tc_tiling_on_sc=False),
        out_type=(
            jax.ShapeDtypeStruct((ROWS, C), F32),
            jax.ShapeDtypeStruct((ROWS, C), F32),
            jax.ShapeDtypeStruct((ROWS, 16), F32),
        ),
        scratch_types=[
            pltpu.VMEM((_CH,), jnp.int32),
            pltpu.VMEM((_CH, C), F32),
            pltpu.VMEM((_CH, C), F32),
            pltpu.VMEM((_CH, 16), F32),
            pltpu.SemaphoreType.DMA,
            pltpu.SemaphoreType.DMA,
            pltpu.SemaphoreType.DMA,
        ],
    )
    def gather_k(idx_hbm, wk_hbm, wv_hbm, pp_hbm, kg_hbm, vg_hbm, pg_hbm,
                 idxb, kb, vb, pb, semk, semv, semp):
        wid = lax.axis_index("s") * 2 + lax.axis_index("c")

        def body(j, carry):
            base = wid * _PER_W + j * _CH
            pltpu.sync_copy(idx_hbm.at[pl.ds(base, _CH)], idxb)
            ck = pltpu.async_copy(wk_hbm.at[idxb], kb, semk)
            cv = pltpu.async_copy(wv_hbm.at[idxb], vb, semv)
            cp = pltpu.async_copy(pp_hbm.at[idxb], pb, semp)
            ck.wait()
            cv.wait()
            cp.wait()
            pltpu.sync_copy(kb, kg_hbm.at[pl.ds(base, _CH)])
            pltpu.sync_copy(vb, vg_hbm.at[pl.ds(base, _CH)])
            pltpu.sync_copy(pb, pg_hbm.at[pl.ds(base, _CH)])
            return carry

        lax.fori_loop(0, _NCH, body, 0)

    return gather_k


def _sc_gather(idxf, wk, wv, pp):
    return _build_sc_gather()(idxf, wk, wv, pp)


_PB = 128
_RB = _PB * K


def _seg_expand(rb, nb):
    ri = lax.broadcasted_iota(jnp.int32, (rb, nb), 0)
    ni = lax.broadcasted_iota(jnp.int32, (rb, nb), 1)
    return jnp.where(ri // K == ni, F32(1.0), F32(0.0))


def _seg_reduce(nb, rb):
    ni = lax.broadcasted_iota(jnp.int32, (nb, rb), 0)
    ri = lax.broadcasted_iota(jnp.int32, (nb, rb), 1)
    return jnp.where(ri // K == ni, F32(1.0), F32(0.0))


def _mom_pad(s0, s1, width):
    z = jnp.zeros((1, 128 - width), F32)
    r0 = jnp.concatenate([s0, z], axis=1)
    r1 = jnp.concatenate([s1, z], axis=1)
    blk = jnp.concatenate([r0, r1, jnp.zeros((6, 128), F32)], axis=0)
    return blk.reshape(1, 8, 128)


def _stats(st_arr, width):
    return st_arr[0:1, 0:width], st_arr[1:2, 0:width]


def _statsfin_body(mom_ref, st_ref):
    moms = jnp.sum(mom_ref[...], axis=0)
    mean = moms[0:1, :] / CNT
    var = moms[1:2, :] / CNT - mean * mean
    rstd = lax.rsqrt(var + EPS)
    st_ref[...] = jnp.concatenate(
        [mean, rstd, jnp.zeros((6, 128), F32)], axis=0)


def _statsfin(mom):
    nblk = mom.shape[0]
    return pl.pallas_call(
        _statsfin_body,
        grid=(1,),
        in_specs=[pl.BlockSpec((nblk, 8, 128), lambda i: (0, 0, 0))],
        out_specs=pl.BlockSpec((8, 128), lambda i: (0, 0)),
        out_shape=jax.ShapeDtypeStruct((8, 128), F32),
    )(mom)


def _pos_h(pg, pp_blk, st, w1p, b1p):
    pp_rep = jnp.dot(st, pp_blk, preferred_element_type=F32)
    rel = pg - pp_rep
    return jnp.dot(rel, w1p, preferred_element_type=F32) + b1p


def _pos_enc(h, mean1, rstd1, g1p, bt1p, w2p, b2p):
    hn = (h - mean1) * rstd1 * g1p + bt1p
    r = jnp.maximum(hn, 0.0)
    return jnp.dot(r, w2p, preferred_element_type=F32) + b2p


def _relmom_body(pg_ref, pp_ref, w1p, b1p, mom_ref):
    st = _seg_expand(_RB, _PB)
    h = _pos_h(pg_ref[...], pp_ref[...], st, w1p[...], b1p[...])
    s0 = jnp.sum(h, axis=0, keepdims=True)
    s1 = jnp.sum(h * h, axis=0, keepdims=True)
    mom_ref[...] = _mom_pad(s0, s1, 16)


def _relmom(pg, pp, w1p, b1p):
    full = lambda shp: pl.BlockSpec(shp, lambda i: (0,) * len(shp))
    nblk = N // _PB
    return pl.pallas_call(
        _relmom_body,
        grid=(nblk,),
        in_specs=[
            pl.BlockSpec((_RB, 16), lambda i: (i, 0)),
            pl.BlockSpec((_PB, 16), lambda i: (i, 0)),
            full((16, 16)), full((1, 16)),
        ],
        out_specs=pl.BlockSpec((1, 8, 128), lambda i: (i, 0, 0)),
        out_shape=jax.ShapeDtypeStruct((nblk, 8, 128), F32),
    )(pg, pp, w1p, b1p)


def _passB_body(kg_ref, pg_ref, wq_ref, pp_ref, st1_ref, w1p, b1p, g1p, bt1p,
                w2p, b2p, attn_ref, mom_ref):
    st = _seg_expand(_RB, _PB)
    mean1, rstd1 = _stats(st1_ref[...], 16)
    h = _pos_h(pg_ref[...], pp_ref[...], st, w1p[...], b1p[...])
    pe = _pos_enc(h, mean1, rstd1, g1p[...], bt1p[...], w2p[...], b2p[...])
    wq_rep = jnp.dot(st, wq_ref[...], preferred_element_type=F32)
    a = kg_ref[...] - wq_rep + pe
    attn_ref[...] = a
    s0 = jnp.sum(a, axis=0, keepdims=True)
    s1 = jnp.sum(a * a, axis=0, keepdims=True)
    mom_ref[...] = _mom_pad(s0, s1, C)


def _passB(kg, pg, wq_full, pp, st1, w1p, b1p, g1p, bt1p, w2p, b2p):
    full = lambda shp: pl.BlockSpec(shp, lambda i: (0,) * len(shp))
    nblk = N // _PB
    return pl.pallas_call(
        _passB_body,
        grid=(nblk,),
        in_specs=[
            pl.BlockSpec((_RB, C), lambda i: (i, 0)),
            pl.BlockSpec((_RB, 16), lambda i: (i, 0)),
            pl.BlockSpec((_PB, C), lambda i: (i, 0)),
            pl.BlockSpec((_PB, 16), lambda i: (i, 0)),
            full((8, 128)),
            full((16, 16)), full((1, 16)), full((1, 16)), full((1, 16)),
            full((16, C)), full((1, C)),
        ],
        out_specs=[
            pl.BlockSpec((_RB, C), lambda i: (i, 0)),
            pl.BlockSpec((1, 8, 128), lambda i: (i, 0, 0)),
        ],
        out_shape=[
            jax.ShapeDtypeStruct((ROWS, C), F32),
            jax.ShapeDtypeStruct((nblk, 8, 128), F32),
        ],
    )(kg, pg, wq_full, pp, st1, w1p, b1p, g1p, bt1p, w2p, b2p)


_RBC = 8192


def _passC_body(attn_ref, st2_ref, w1T, b1m, g1m, bt1m, s_ref, mom_ref):
    mean2, rstd2 = _stats(st2_ref[...], C)
    an = (attn_ref[...] - mean2) * rstd2 * g1m[...] + bt1m[...]
    r = jnp.maximum(an, 0.0)
    s = jnp.dot(r, w1T[...], preferred_element_type=F32) + b1m[...]
    s_ref[...] = s
    s0 = jnp.sum(s, axis=0, keepdims=True)
    s1 = jnp.sum(s * s, axis=0, keepdims=True)
    mom_ref[...] = _mom_pad(s0, s1, DH)


def _passC(attn, st2, w1T, b1m, g1m, bt1m):
    full = lambda shp: pl.BlockSpec(shp, lambda i: (0,) * len(shp))
    nblk = ROWS // _RBC
    return pl.pallas_call(
        _passC_body,
        grid=(nblk,),
        in_specs=[
            pl.BlockSpec((_RBC, C), lambda i: (i, 0)),
            full((8, 128)),
            full((C, DH)), full((1, DH)), full((1, C)), full((1, C)),
        ],
        out_specs=[
            pl.BlockSpec((_RBC, DH), lambda i: (i, 0)),
            pl.BlockSpec((1, 8, 128), lambda i: (i, 0, 0)),
        ],
        out_shape=[
            jax.ShapeDtypeStruct((ROWS, DH), F32),
            jax.ShapeDtypeStruct((nblk, 8, 128), F32),
        ],
    )(attn, st2, w1T, b1m, g1m, bt1m)


def _passD_body(s_ref, st3_ref, vg_ref, pg_ref, pp_ref, st1_ref,
                w1p, b1p, g1p, bt1p, w2p, b2p,
                w2T, b2m, g2m, bt2m, t8, out_ref):
    mean3, rstd3 = _stats(st3_ref[...], DH)
    sn = (s_ref[...] - mean3) * rstd3 * g2m[...] + bt2m[...]
    u = jnp.dot(jnp.maximum(sn, 0.0), w2T[...],
                preferred_element_type=F32) + b2m[...]
    e = jnp.exp(u)
    ssel = _seg_reduce(_PB, _RB)
    st = _seg_expand(_RB, _PB)
    den = jnp.dot(ssel, e, preferred_element_type=F32)
    den_rep = jnp.dot(st, den, preferred_element_type=F32)
    w = e / den_rep
    mean1, rstd1 = _stats(st1_ref[...], 16)
    h = _pos_h(pg_ref[...], pp_ref[...], st, w1p[...], b1p[...])
    pe = _pos_enc(h, mean1, rstd1, g1p[...], bt1p[...], w2p[...], b2p[...])
    v = vg_ref[...] + pe
    w64 = jnp.dot(w, t8[...], preferred_element_type=F32)
    out_ref[...] = jnp.dot(ssel, v * w64,
                           preferred_element_type=F32)


def _passD(s, st3, vg, pg, pp, st1, w1p, b1p, g1p, bt1p, w2p, b2p,
           w2T, b2m, g2m, bt2m, t8):
    full = lambda shp: pl.BlockSpec(shp, lambda i: (0,) * len(shp))
    nblk = N // _PB
    return pl.pallas_call(
        _passD_body,
        grid=(nblk,),
        in_specs=[
            pl.BlockSpec((_RB, DH), lambda i: (i, 0)),
            full((8, 128)),
            pl.BlockSpec((_RB, C), lambda i: (i, 0)),
            pl.BlockSpec((_RB, 16), lambda i: (i, 0)),
            pl.BlockSpec((_PB, 16), lambda i: (i, 0)),
            full((8, 128)),
            full((16, 16)), full((1, 16)), full((1, 16)), full((1, 16)),
            full((16, C)), full((1, C)),
            full((DH, DH)), full((1, DH)), full((1, DH)), full((1, DH)),
            full((DH, C)),
        ],
        out_specs=pl.BlockSpec((_PB, C), lambda i: (i, 0)),
        out_shape=jax.ShapeDtypeStruct((N, C), F32),
    )(s, st3, vg, pg, pp, st1, w1p, b1p, g1p, bt1p, w2p, b2p,
      w2T, b2m, g2m, bt2m, t8)


def kernel(pos, x, offset, wq, bq, wk, bk, wv, bv, pos_w1, pos_b1, pos_bn_g,
           pos_bn_b, pos_w2, pos_b2, mlp_bn1_g, mlp_bn1_b, mlp_w1, mlp_b1,
           mlp_bn2_g, mlp_bn2_b, mlp_w2, mlp_b2):
    posT = pos.T
    pp = jnp.pad(pos, ((0, 0), (0, 13)))
    pad16 = lambda v: jnp.pad(v, (0, 13)).reshape(1, 16)
    w1p = jnp.pad(pos_w1.T, ((0, 13), (0, 13)))
    b1p = pad16(pos_b1)
    g1p = pad16(pos_bn_g)
    bt1p = pad16(pos_bn_b)
    w2p = jnp.pad(pos_w2.T, ((0, 13), (0, 0)))
    b2p = pos_b2.reshape(1, C)
    w1T = mlp_w1.T
    b1m = mlp_b1.reshape(1, DH)
    g1m = mlp_bn1_g.reshape(1, C)
    bt1m = mlp_bn1_b.reshape(1, C)
    w2T = mlp_w2.T
    b2m = mlp_b2.reshape(1, DH)
    g2m = mlp_bn2_g.reshape(1, DH)
    bt2m = mlp_bn2_b.reshape(1, DH)
    t8 = jnp.tile(jnp.eye(DH, dtype=F32), (1, H))

    wq_full, wk_full, wv_full = _qkv(x, wq.T, bq.reshape(1, C), wk.T,
                                     bk.reshape(1, C), wv.T, bv.reshape(1, C))
    idx = _knn(pos, posT)
    kg, vg, pg = _sc_gather(idx.reshape(ROWS), wk_full, wv_full, pp)
    st1 = _statsfin(_relmom(pg, pp, w1p, b1p))
    attn, mom2 = _passB(kg, pg, wq_full, pp, st1, w1p, b1p, g1p, bt1p,
                        w2p, b2p)
    s, mom3 = _passC(attn, _statsfin(mom2), w1T, b1m, g1m, bt1m)
    return _passD(s, _statsfin(mom3), vg, pg, pp, st1, w1p, b1p, g1p, bt1p,
                  w2p, b2p, w2T, b2m, g2m, bt2m, t8)
